# baseline (speedup 1.0000x reference)
import functools

import jax
import jax.numpy as jnp
from jax import lax
from jax.experimental import pallas as pl
from jax.experimental.pallas import tpu as pltpu
from jax.experimental.pallas import tpu_sc as plsc

L = 16
W = 128


def kernel(users, jokes, mu_U, logvar_U, mu_V, logvar_V, z_U, z_V):
    B = users.shape[0]
    K = mu_U.shape[1]
    R = W // K
    info = plsc.get_sparse_core_info()
    NC, NS = info.num_cores, info.num_subcores
    NW = NC * NS
    BPW = B // NW
    CH = 128
    NCH = BPW // CH
    ZRW = BPW * K // W

    mu_U = mu_U.reshape(-1, W)
    logvar_U = logvar_U.reshape(-1, W)
    mu_V = mu_V.reshape(-1, W)
    logvar_V = logvar_V.reshape(-1, W)
    z_U = z_U.reshape(-1, W)
    z_V = z_V.reshape(-1, W)

    mesh = plsc.VectorSubcoreMesh(core_axis_name="c", subcore_axis_name="s")

    @functools.partial(
        pl.kernel,
        mesh=mesh,
        compiler_params=pltpu.CompilerParams(
            needs_layout_passes=False, use_tc_tiling_on_sc=False),
        out_type=jax.ShapeDtypeStruct((B,), jnp.float32),
        scratch_types=[
            pltpu.VMEM((BPW,), jnp.int32),
            pltpu.VMEM((BPW,), jnp.int32),
            pltpu.VMEM((NCH, CH), jnp.int32),
            pltpu.VMEM((NCH, CH), jnp.int32),
            pltpu.VMEM((CH, W), jnp.float32),
            pltpu.VMEM((CH, W), jnp.float32),
            pltpu.VMEM((CH, W), jnp.float32),
            pltpu.VMEM((CH, W), jnp.float32),
            pltpu.VMEM((ZRW, W), jnp.float32),
            pltpu.VMEM((ZRW, W), jnp.float32),
            pltpu.VMEM((BPW,), jnp.float32),
            pltpu.SemaphoreType.DMA,
        ],
    )
    def run(users_h, jokes_h, mu_u_h, lv_u_h, mu_v_h, lv_v_h, zu_h, zv_h,
            out_h, raw_u, raw_v, rows_u, rows_v, t_mu_u, t_lv_u, t_mu_v,
            t_lv_v, b_zu, b_zv, outv, sem):
        wid = lax.axis_index("s") * NC + lax.axis_index("c")
        base = wid * BPW

        pltpu.sync_copy(users_h.at[pl.ds(base, BPW)], raw_u)
        pltpu.sync_copy(jokes_h.at[pl.ds(base, BPW)], raw_v)

        def make_rows(i, carry):
            c = i // (CH // L)
            j = i % (CH // L)
            sl = pl.ds(c * CH + j * L, L)
            rows_u[c, pl.ds(j * L, L)] = raw_u[sl] // R
            rows_v[c, pl.ds(j * L, L)] = raw_v[sl] // R
            return carry

        lax.fori_loop(0, BPW // L, make_rows, 0)

        pltpu.sync_copy(zu_h.at[pl.ds(wid * ZRW, ZRW)], b_zu)
        pltpu.sync_copy(zv_h.at[pl.ds(wid * ZRW, ZRW)], b_zv)

        lane = lax.iota(jnp.int32, L)
        zc0 = (lane % R) * K
        zr_off = lane // R

        for c in range(NCH):
            cp1 = pltpu.async_copy(mu_u_h.at[rows_u.at[c]], t_mu_u, sem)
            cp2 = pltpu.async_copy(lv_u_h.at[rows_u.at[c]], t_lv_u, sem)
            cp3 = pltpu.async_copy(mu_v_h.at[rows_v.at[c]], t_mu_v, sem)
            cp4 = pltpu.async_copy(lv_v_h.at[rows_v.at[c]], t_lv_v, sem)
            cp1.wait()
            cp2.wait()
            cp3.wait()
            cp4.wait()

            def group(g, carry, c=c):
                b16 = g * L + lane
                sl = pl.ds(c * CH + g * L, L)
                cu0 = (raw_u[sl] % R) * K
                cv0 = (raw_v[sl] % R) * K
                zr = c * (CH * K // W) + g * (L * K // W) + zr_off
                acc = jnp.zeros((L,), jnp.float32)
                for k in range(K):
                    mu = plsc.load_gather(t_mu_u, [b16, cu0 + k])
                    lvu = plsc.load_gather(t_lv_u, [b16, cu0 + k])
                    mv = plsc.load_gather(t_mu_v, [b16, cv0 + k])
                    lvv = plsc.load_gather(t_lv_v, [b16, cv0 + k])
                    zu = plsc.load_gather(b_zu, [zr, zc0 + k])
                    zv = plsc.load_gather(b_zv, [zr, zc0 + k])
                    u = zu * jnp.exp(lvu * 0.5) + mu
                    v = zv * jnp.exp(lvv * 0.5) + mv
                    acc = acc + u * v
                outv[pl.ds(c * CH + g * L, L)] = acc
                return carry

            lax.fori_loop(0, CH // L, group, 0)

        pltpu.sync_copy(outv, out_h.at[pl.ds(base, BPW)])

    return run(users, jokes, mu_U, logvar_U, mu_V, logvar_V, z_U, z_V)
